# baseline (speedup 1.0000x reference)
import functools
import jax
import jax.numpy as jnp
from jax import lax
from jax.experimental import pallas as pl
from jax.experimental.pallas import tpu as pltpu
from jax.experimental.pallas import tpu_sc as plsc

_N = 10000
_NPAD = 10240
_D = 128
_EPAD = 327680
_NCHUNK = 32
_BS = 128
_NBLK = 80
_SBLK = 16
_NROUND = _NBLK // _SBLK
_ROWS_PER_TILE = _NPAD // 16


def _zeros16():
    return jnp.zeros((16,), jnp.float32)


def _deg_body(row_hbm, ew_hbm, out_hbm, row_v, ew_v, deg_v, deg_red, deg_tmp,
              deg_stage):
    c = lax.axis_index("c")
    s = lax.axis_index("s")
    wid = s * 2 + c
    my_rows = s * _ROWS_PER_TILE

    def zero_deg(i, _):
        deg_v[pl.ds(i * 16, 16)] = _zeros16()
        return 0
    lax.fori_loop(0, _NPAD // 16, zero_deg, 0)

    pltpu.sync_copy(row_hbm.at[wid], row_v)
    pltpu.sync_copy(ew_hbm.at[wid], ew_v)

    def deg_step(jj, _):
        for k in range(8):
            idx = row_v[jj, pl.ds(k * 16, 16)]
            w = ew_v[jj, pl.ds(k * 16, 16)]
            plsc.addupdate_scatter(deg_v, [idx], w)
        return 0
    lax.fori_loop(0, _NBLK, deg_step, 0)

    pltpu.sync_copy(deg_v, deg_stage.at[s])
    plsc.subcore_barrier()

    pltpu.sync_copy(deg_stage.at[0, pl.ds(my_rows, _ROWS_PER_TILE)], deg_red)
    for p in range(1, 16):
        pltpu.sync_copy(deg_stage.at[p, pl.ds(my_rows, _ROWS_PER_TILE)],
                        deg_tmp)

        def add_step(i, _):
            deg_red[pl.ds(i * 16, 16)] = (deg_red[pl.ds(i * 16, 16)]
                                          + deg_tmp[pl.ds(i * 16, 16)])
            return 0
        lax.fori_loop(0, _ROWS_PER_TILE // 16, add_step, 0)

    pltpu.sync_copy(deg_red, out_hbm.at[c, pl.ds(my_rows, _ROWS_PER_TILE)])


@jax.jit
def _sc_degree(row3, ew3):
    mesh = plsc.VectorSubcoreMesh(core_axis_name="c", subcore_axis_name="s",
                                  num_cores=2)
    f = functools.partial(
        pl.kernel,
        mesh=mesh,
        compiler_params=pltpu.CompilerParams(needs_layout_passes=False),
        out_type=jax.ShapeDtypeStruct((2, _NPAD), jnp.float32),
        scratch_types=[
            pltpu.VMEM((_NBLK, _BS), jnp.int32),
            pltpu.VMEM((_NBLK, _BS), jnp.float32),
            pltpu.VMEM((_NPAD,), jnp.float32),
            pltpu.VMEM((_ROWS_PER_TILE,), jnp.float32),
            pltpu.VMEM((_ROWS_PER_TILE,), jnp.float32),
            pltpu.VMEM_SHARED((16, _NPAD), jnp.float32),
        ],
    )(_deg_body)
    return f(row3, ew3)


_B0 = 128
_B1 = 32


def _agg_body(xs_hbm, row_hbm, col_hbm, ew_hbm, out_hbm,
              row_v, col_v, ew_v, rows_buf, tx1_acc, g0, g1, s0, s1, st):
    gsem = (g0, g1)
    ssem = (s0, s1)
    c = lax.axis_index("c")
    s = lax.axis_index("s")
    base = jnp.where(c == 0, s * _B0, 16 * _B0 + s * _B1)
    my_rows = s * _ROWS_PER_TILE

    def stage_start(roff, dset):
        roff = pl.multiple_of(roff, 8)
        pltpu.make_async_copy(row_hbm.at[pl.ds(roff, _SBLK)],
                              row_v.at[dset], st).start()
        pltpu.make_async_copy(col_hbm.at[pl.ds(roff, _SBLK)],
                              col_v.at[dset], st).start()
        pltpu.make_async_copy(ew_hbm.at[pl.ds(roff, _SBLK)],
                              ew_v.at[dset], st).start()

    def stage_wait(dset):
        pltpu.make_async_copy(row_hbm.at[pl.ds(0, _SBLK)],
                              row_v.at[dset], st).wait()
        pltpu.make_async_copy(col_hbm.at[pl.ds(0, _SBLK)],
                              col_v.at[dset], st).wait()
        pltpu.make_async_copy(ew_hbm.at[pl.ds(0, _SBLK)],
                              ew_v.at[dset], st).wait()

    stage_start(base, 0)

    def zero_buf(i, _):
        for k in range(8):
            rows_buf[0, i, pl.ds(k * 16, 16)] = _zeros16()
        return 0
    lax.fori_loop(0, _BS, zero_buf, 0)

    def zero_acc(j, _):
        pltpu.sync_copy(rows_buf.at[0],
                        tx1_acc.at[pl.ds(my_rows + j * _BS, _BS)])
        return 0
    lax.fori_loop(0, _ROWS_PER_TILE // _BS, zero_acc, 0)
    plsc.subcore_barrier()

    def gather_start(dset, b, q):
        for h in range(4):
            sub = _BS // 4
            pltpu.make_async_copy(
                xs_hbm.at[row_v.at[dset, b, pl.ds(h * sub, sub)]],
                rows_buf.at[q, pl.ds(h * sub, sub)], gsem[q]).start()

    def gather_wait(dset, b, q):
        pltpu.make_async_copy(xs_hbm.at[row_v.at[dset, b]], rows_buf.at[q],
                              gsem[q]).wait()

    def scatter_start(dset, q, b):
        pltpu.make_async_copy(rows_buf.at[q], tx1_acc.at[col_v.at[dset, b]],
                              ssem[q]).start(add=True)

    def scatter_wait(dset, q, b):
        pltpu.make_async_copy(rows_buf.at[q], tx1_acc.at[col_v.at[dset, b]],
                              ssem[q]).wait()

    def scale(dset, q, b):
        def scale_step(jj, _):
            wb = plsc.load_gather(
                ew_v, [jnp.full((16,), dset, jnp.int32),
                       jnp.full((16,), b, jnp.int32),
                       jnp.full((16,), jj, jnp.int32)])
            for k in range(8):
                rows_buf[q, jj, pl.ds(k * 16, 16)] = (
                    rows_buf[q, jj, pl.ds(k * 16, 16)] * wb)
            return 0
        lax.fori_loop(0, _BS, scale_step, 0)

    def do_round(dset):
        gather_start(dset, 0, 0)

        def ring_step(i, _):
            for q in range(2):
                b = i * 2 + q
                gather_wait(dset, b, q)
                if q == 0:
                    @pl.when(i > 0)
                    def _():
                        scatter_wait(dset, 1, b)
                    gather_start(dset, b + 1, 1)
                else:
                    @pl.when(i < _SBLK // 2 - 1)
                    def _():
                        scatter_wait(dset, 0, b)
                        gather_start(dset, b + 1, 0)
                scale(dset, q, b)
                scatter_start(dset, q, b)
            return 0
        lax.fori_loop(0, _SBLK // 2, ring_step, 0)

        scatter_wait(dset, 0, 0)
        scatter_wait(dset, 1, 0)

    _R0N = _B0 // _SBLK
    _R1N = _B1 // _SBLK
    for r in range(_R0N):
        dset = r % 2
        if r < _R1N:
            stage_wait(dset)
            if r + 1 < _R1N:
                stage_start(base + (r + 1) * _SBLK, 1 - dset)
            else:
                @pl.when(c == 0)
                def _():
                    stage_start(base + (r + 1) * _SBLK, 1 - dset)
            do_round(dset)
        else:
            @pl.when(c == 0)
            def _():
                stage_wait(dset)
                if r + 1 < _R0N:
                    stage_start(base + (r + 1) * _SBLK, 1 - dset)
                do_round(dset)

    plsc.subcore_barrier()
    pltpu.sync_copy(tx1_acc.at[pl.ds(my_rows, _ROWS_PER_TILE)],
                    out_hbm.at[c, pl.ds(my_rows, _ROWS_PER_TILE)])


@jax.jit
def _sc_aggregate(xs, row2, col2, ew2):
    mesh = plsc.VectorSubcoreMesh(core_axis_name="c", subcore_axis_name="s",
                                  num_cores=2)
    f = functools.partial(
        pl.kernel,
        mesh=mesh,
        compiler_params=pltpu.CompilerParams(needs_layout_passes=False),
        out_type=jax.ShapeDtypeStruct((2, _NPAD, _D), jnp.float32),
        scratch_types=[
            pltpu.VMEM((2, _SBLK, _BS), jnp.int32),
            pltpu.VMEM((2, _SBLK, _BS), jnp.int32),
            pltpu.VMEM((2, _SBLK, _BS), jnp.float32),
            pltpu.VMEM((2, _BS, _D), jnp.float32),
            pltpu.VMEM_SHARED((_NPAD, _D), jnp.float32),
        ] + [pltpu.SemaphoreType.DMA] * 5,
    )(_agg_body)
    return f(xs, row2, col2, ew2)


_BLK = 1024


def _prescale_body(x_ref, degp_ref, xs_ref):
    d = degp_ref[:, 0:1] + degp_ref[:, 1:2]
    dinv = jnp.where(d > 0.0, lax.rsqrt(jnp.where(d > 0.0, d, 1.0)), 0.0)
    xs_ref[...] = x_ref[...] * dinv


@jax.jit
def _prescale(xpad, degp_t):
    grid = (_NPAD // _BLK,)
    row_blk = lambda i: (i, 0)
    return pl.pallas_call(
        _prescale_body,
        grid=grid,
        in_specs=[
            pl.BlockSpec((_BLK, _D), row_blk),
            pl.BlockSpec((_BLK, 2), row_blk),
        ],
        out_specs=pl.BlockSpec((_BLK, _D), row_blk),
        out_shape=jax.ShapeDtypeStruct((_NPAD, _D), jnp.float32),
    )(xpad, degp_t)


def _dense_body(x_ref, t0_ref, t1_ref, degp_ref, wz0_ref, wz1_ref, wh0_ref,
                wh1_ref, bz_ref, bh_ref, wl_ref, bl_ref, out_ref):
    x = x_ref[...]
    d = degp_ref[:, 0:1] + degp_ref[:, 1:2]
    dinv = jnp.where(d > 0.0, lax.rsqrt(jnp.where(d > 0.0, d, 1.0)), 0.0)
    t = (t0_ref[...] + t1_ref[...]) * (-dinv)
    zp = (jnp.dot(x, wz0_ref[...], preferred_element_type=jnp.float32)
          + jnp.dot(t, wz1_ref[...], preferred_element_type=jnp.float32)
          + bz_ref[...])
    hp = (jnp.dot(x, wh0_ref[...], preferred_element_type=jnp.float32)
          + jnp.dot(t, wh1_ref[...], preferred_element_type=jnp.float32)
          + bh_ref[...])
    z = jax.nn.sigmoid(zp)
    ht = jnp.tanh(hp)
    h = jnp.maximum((1.0 - z) * ht, 0.0)
    out_ref[...] = jnp.dot(h, wl_ref[...], preferred_element_type=jnp.float32) + bl_ref[...]


@jax.jit
def _dense(xpad, t0, t1, degp_t, wz0, wz1, wh0, wh1, bz, bh, wl, bl):
    grid = (_NPAD // _BLK,)
    full = lambda i: (0, 0)
    row_blk = lambda i: (i, 0)
    return pl.pallas_call(
        _dense_body,
        grid=grid,
        in_specs=[
            pl.BlockSpec((_BLK, _D), row_blk),
            pl.BlockSpec((_BLK, _D), row_blk),
            pl.BlockSpec((_BLK, _D), row_blk),
            pl.BlockSpec((_BLK, 2), row_blk),
            pl.BlockSpec((_D, _D), full),
            pl.BlockSpec((_D, _D), full),
            pl.BlockSpec((_D, _D), full),
            pl.BlockSpec((_D, _D), full),
            pl.BlockSpec((1, _D), full),
            pl.BlockSpec((1, _D), full),
            pl.BlockSpec((_D, 1), full),
            pl.BlockSpec((1, 1), full),
        ],
        out_specs=pl.BlockSpec((_BLK, 1), row_blk),
        out_shape=jax.ShapeDtypeStruct((_NPAD, 1), jnp.float32),
    )(xpad, t0, t1, degp_t, wz0, wz1, wh0, wh1, bz, bh, wl, bl)


def kernel(x, edge_weight, Wxz0, Wxz1, bxz, Whz0, Whz1, bhz, Wxr0, Wxr1, bxr,
           Whr0, Whr1, bhr, Wxh0, Wxh1, bxh, Whh0, Whh1, bhh, Wl, bl,
           edge_index):
    e = edge_weight.shape[0]
    pad = _EPAD - e
    row = jnp.concatenate([edge_index[0], jnp.zeros((pad,), jnp.int32)])
    col = jnp.concatenate([edge_index[1], jnp.zeros((pad,), jnp.int32)])
    ew = jnp.concatenate([edge_weight, jnp.zeros((pad,), jnp.float32)])
    row3 = row.reshape(_NCHUNK, _NBLK, _BS)
    col3 = col.reshape(_NCHUNK, _NBLK, _BS)
    ew3 = ew.reshape(_NCHUNK, _NBLK, _BS)
    xpad = jnp.pad(x, ((0, _NPAD - x.shape[0]), (0, 0)))

    degp = _sc_degree(row3, ew3)
    degp_t = degp.T
    xs = _prescale(xpad, degp_t)
    parts = _sc_aggregate(xs, row.reshape(-1, _BS), col.reshape(-1, _BS),
                          ew.reshape(-1, _BS))

    bz = (bxz + bhz).reshape(1, -1)
    bh = (bxh + bhh).reshape(1, -1)
    out = _dense(xpad, parts[0], parts[1], degp_t, Wxz0, Wxz1, Wxh0, Wxh1,
                 bz, bh, Wl, bl.reshape(1, 1))
    return out[:_N]


# speedup vs baseline: 1.3072x; 1.3072x over previous
import functools
import jax
import jax.numpy as jnp
from jax import lax
from jax.experimental import pallas as pl
from jax.experimental.pallas import tpu as pltpu
from jax.experimental.pallas import tpu_sc as plsc

_N = 10000
_NPAD = 10240
_D = 128
_EPAD = 327680
_NCHUNK = 32
_BS = 128
_NBLK = 80
_SBLK = 16
_NROUND = _NBLK // _SBLK
_ROWS_PER_TILE = _NPAD // 16


def _zeros16():
    return jnp.zeros((16,), jnp.float32)


def _deg_body(row_hbm, ew_hbm, out_hbm, row_v, ew_v, deg_v, deg_red, deg_tmp,
              deg_stage):
    c = lax.axis_index("c")
    s = lax.axis_index("s")
    wid = s * 2 + c
    my_rows = s * _ROWS_PER_TILE

    def zero_deg(i, _):
        deg_v[pl.ds(i * 16, 16)] = _zeros16()
        return 0
    lax.fori_loop(0, _NPAD // 16, zero_deg, 0)

    pltpu.sync_copy(row_hbm.at[wid], row_v)
    pltpu.sync_copy(ew_hbm.at[wid], ew_v)

    def deg_step(jj, _):
        for k in range(8):
            idx = row_v[jj, pl.ds(k * 16, 16)]
            w = ew_v[jj, pl.ds(k * 16, 16)]
            plsc.addupdate_scatter(deg_v, [idx], w)
        return 0
    lax.fori_loop(0, _NBLK, deg_step, 0)

    pltpu.sync_copy(deg_v, deg_stage.at[s])
    plsc.subcore_barrier()

    pltpu.sync_copy(deg_stage.at[0, pl.ds(my_rows, _ROWS_PER_TILE)], deg_red)
    for p in range(1, 16):
        pltpu.sync_copy(deg_stage.at[p, pl.ds(my_rows, _ROWS_PER_TILE)],
                        deg_tmp)

        def add_step(i, _):
            deg_red[pl.ds(i * 16, 16)] = (deg_red[pl.ds(i * 16, 16)]
                                          + deg_tmp[pl.ds(i * 16, 16)])
            return 0
        lax.fori_loop(0, _ROWS_PER_TILE // 16, add_step, 0)

    pltpu.sync_copy(deg_red, out_hbm.at[c, pl.ds(my_rows, _ROWS_PER_TILE)])


@jax.jit
def _sc_degree(row3, ew3):
    mesh = plsc.VectorSubcoreMesh(core_axis_name="c", subcore_axis_name="s",
                                  num_cores=2)
    f = functools.partial(
        pl.kernel,
        mesh=mesh,
        compiler_params=pltpu.CompilerParams(needs_layout_passes=False),
        out_type=jax.ShapeDtypeStruct((2, _NPAD), jnp.float32),
        scratch_types=[
            pltpu.VMEM((_NBLK, _BS), jnp.int32),
            pltpu.VMEM((_NBLK, _BS), jnp.float32),
            pltpu.VMEM((_NPAD,), jnp.float32),
            pltpu.VMEM((_ROWS_PER_TILE,), jnp.float32),
            pltpu.VMEM((_ROWS_PER_TILE,), jnp.float32),
            pltpu.VMEM_SHARED((16, _NPAD), jnp.float32),
        ],
    )(_deg_body)
    return f(row3, ew3)


_B0 = 144
_B1 = 16


def _agg_body(xs_hbm, row_hbm, col_hbm, ew_hbm, out_hbm,
              row_v, col_v, ew_v, rows_buf, tx1_acc, g0, g1, s0, s1, st):
    gsem = (g0, g1)
    ssem = (s0, s1)
    c = lax.axis_index("c")
    s = lax.axis_index("s")
    base = jnp.where(c == 0, s * _B0, 16 * _B0 + s * _B1)
    my_rows = s * _ROWS_PER_TILE

    def stage_start(roff, dset):
        roff = pl.multiple_of(roff, 8)
        pltpu.make_async_copy(row_hbm.at[pl.ds(roff, _SBLK)],
                              row_v.at[dset], st).start()
        pltpu.make_async_copy(col_hbm.at[pl.ds(roff, _SBLK)],
                              col_v.at[dset], st).start()
        pltpu.make_async_copy(ew_hbm.at[pl.ds(roff, _SBLK)],
                              ew_v.at[dset], st).start()

    def stage_wait(dset):
        pltpu.make_async_copy(row_hbm.at[pl.ds(0, _SBLK)],
                              row_v.at[dset], st).wait()
        pltpu.make_async_copy(col_hbm.at[pl.ds(0, _SBLK)],
                              col_v.at[dset], st).wait()
        pltpu.make_async_copy(ew_hbm.at[pl.ds(0, _SBLK)],
                              ew_v.at[dset], st).wait()

    stage_start(base, 0)

    def zero_buf(i, _):
        for k in range(8):
            rows_buf[0, i, pl.ds(k * 16, 16)] = _zeros16()
        return 0
    lax.fori_loop(0, _BS, zero_buf, 0)

    def zero_acc(j, _):
        pltpu.sync_copy(rows_buf.at[0],
                        tx1_acc.at[pl.ds(my_rows + j * _BS, _BS)])
        return 0
    lax.fori_loop(0, _ROWS_PER_TILE // _BS, zero_acc, 0)
    plsc.subcore_barrier()

    def gather_start(dset, b, q):
        for h in range(4):
            sub = _BS // 4
            pltpu.make_async_copy(
                xs_hbm.at[row_v.at[dset, b, pl.ds(h * sub, sub)]],
                rows_buf.at[q, pl.ds(h * sub, sub)], gsem[q]).start()

    def gather_wait(dset, b, q):
        pltpu.make_async_copy(xs_hbm.at[row_v.at[dset, b]], rows_buf.at[q],
                              gsem[q]).wait()

    def scatter_start(dset, q, b):
        pltpu.make_async_copy(rows_buf.at[q], tx1_acc.at[col_v.at[dset, b]],
                              ssem[q]).start(add=True)

    def scatter_wait(dset, q, b):
        pltpu.make_async_copy(rows_buf.at[q], tx1_acc.at[col_v.at[dset, b]],
                              ssem[q]).wait()

    def scale(dset, q, b):
        def scale_step(jj, _):
            wb = plsc.load_gather(
                ew_v, [jnp.full((16,), dset, jnp.int32),
                       jnp.full((16,), b, jnp.int32),
                       jnp.full((16,), jj, jnp.int32)])
            for k in range(8):
                rows_buf[q, jj, pl.ds(k * 16, 16)] = (
                    rows_buf[q, jj, pl.ds(k * 16, 16)] * wb)
            return 0
        lax.fori_loop(0, _BS, scale_step, 0)

    def do_round(dset):
        gather_start(dset, 0, 0)

        def ring_step(i, _):
            for q in range(2):
                b = i * 2 + q
                gather_wait(dset, b, q)
                if q == 0:
                    @pl.when(i > 0)
                    def _():
                        scatter_wait(dset, 1, b)
                    gather_start(dset, b + 1, 1)
                else:
                    @pl.when(i < _SBLK // 2 - 1)
                    def _():
                        scatter_wait(dset, 0, b)
                        gather_start(dset, b + 1, 0)
                scale(dset, q, b)
                scatter_start(dset, q, b)
            return 0
        lax.fori_loop(0, _SBLK // 2, ring_step, 0)

        scatter_wait(dset, 0, 0)
        scatter_wait(dset, 1, 0)

    _R0N = _B0 // _SBLK
    _R1N = _B1 // _SBLK
    for r in range(_R0N):
        dset = r % 2
        if r < _R1N:
            stage_wait(dset)
            if r + 1 < _R1N:
                stage_start(base + (r + 1) * _SBLK, 1 - dset)
            else:
                @pl.when(c == 0)
                def _():
                    stage_start(base + (r + 1) * _SBLK, 1 - dset)
            do_round(dset)
        else:
            @pl.when(c == 0)
            def _():
                stage_wait(dset)
                if r + 1 < _R0N:
                    stage_start(base + (r + 1) * _SBLK, 1 - dset)
                do_round(dset)

    plsc.subcore_barrier()
    pltpu.sync_copy(tx1_acc.at[pl.ds(my_rows, _ROWS_PER_TILE)],
                    out_hbm.at[c, pl.ds(my_rows, _ROWS_PER_TILE)])


@jax.jit
def _sc_aggregate(xs, row2, col2, ew2):
    mesh = plsc.VectorSubcoreMesh(core_axis_name="c", subcore_axis_name="s",
                                  num_cores=2)
    f = functools.partial(
        pl.kernel,
        mesh=mesh,
        compiler_params=pltpu.CompilerParams(needs_layout_passes=False),
        out_type=jax.ShapeDtypeStruct((2, _NPAD, _D), jnp.float32),
        scratch_types=[
            pltpu.VMEM((2, _SBLK, _BS), jnp.int32),
            pltpu.VMEM((2, _SBLK, _BS), jnp.int32),
            pltpu.VMEM((2, _SBLK, _BS), jnp.float32),
            pltpu.VMEM((2, _BS, _D), jnp.float32),
            pltpu.VMEM_SHARED((_NPAD, _D), jnp.float32),
        ] + [pltpu.SemaphoreType.DMA] * 5,
    )(_agg_body)
    return f(xs, row2, col2, ew2)


_BLK = 1024


def _prescale_body(x_ref, degp_ref, xs_ref):
    d = degp_ref[:, 0:1] + degp_ref[:, 1:2]
    dinv = jnp.where(d > 0.0, lax.rsqrt(jnp.where(d > 0.0, d, 1.0)), 0.0)
    xs_ref[...] = x_ref[...] * dinv


@jax.jit
def _prescale(xpad, degp_t):
    grid = (_NPAD // _BLK,)
    row_blk = lambda i: (i, 0)
    return pl.pallas_call(
        _prescale_body,
        grid=grid,
        in_specs=[
            pl.BlockSpec((_BLK, _D), row_blk),
            pl.BlockSpec((_BLK, 2), row_blk),
        ],
        out_specs=pl.BlockSpec((_BLK, _D), row_blk),
        out_shape=jax.ShapeDtypeStruct((_NPAD, _D), jnp.float32),
    )(xpad, degp_t)


def _dense_body(x_ref, t0_ref, t1_ref, degp_ref, wz0_ref, wz1_ref, wh0_ref,
                wh1_ref, bz_ref, bh_ref, wl_ref, bl_ref, out_ref):
    x = x_ref[...]
    d = degp_ref[:, 0:1] + degp_ref[:, 1:2]
    dinv = jnp.where(d > 0.0, lax.rsqrt(jnp.where(d > 0.0, d, 1.0)), 0.0)
    t = (t0_ref[...] + t1_ref[...]) * (-dinv)
    zp = (jnp.dot(x, wz0_ref[...], preferred_element_type=jnp.float32)
          + jnp.dot(t, wz1_ref[...], preferred_element_type=jnp.float32)
          + bz_ref[...])
    hp = (jnp.dot(x, wh0_ref[...], preferred_element_type=jnp.float32)
          + jnp.dot(t, wh1_ref[...], preferred_element_type=jnp.float32)
          + bh_ref[...])
    z = jax.nn.sigmoid(zp)
    ht = jnp.tanh(hp)
    h = jnp.maximum((1.0 - z) * ht, 0.0)
    out_ref[...] = jnp.dot(h, wl_ref[...], preferred_element_type=jnp.float32) + bl_ref[...]


@jax.jit
def _dense(xpad, t0, t1, degp_t, wz0, wz1, wh0, wh1, bz, bh, wl, bl):
    grid = (_NPAD // _BLK,)
    full = lambda i: (0, 0)
    row_blk = lambda i: (i, 0)
    return pl.pallas_call(
        _dense_body,
        grid=grid,
        in_specs=[
            pl.BlockSpec((_BLK, _D), row_blk),
            pl.BlockSpec((_BLK, _D), row_blk),
            pl.BlockSpec((_BLK, _D), row_blk),
            pl.BlockSpec((_BLK, 2), row_blk),
            pl.BlockSpec((_D, _D), full),
            pl.BlockSpec((_D, _D), full),
            pl.BlockSpec((_D, _D), full),
            pl.BlockSpec((_D, _D), full),
            pl.BlockSpec((1, _D), full),
            pl.BlockSpec((1, _D), full),
            pl.BlockSpec((_D, 1), full),
            pl.BlockSpec((1, 1), full),
        ],
        out_specs=pl.BlockSpec((_BLK, 1), row_blk),
        out_shape=jax.ShapeDtypeStruct((_NPAD, 1), jnp.float32),
    )(xpad, t0, t1, degp_t, wz0, wz1, wh0, wh1, bz, bh, wl, bl)


def kernel(x, edge_weight, Wxz0, Wxz1, bxz, Whz0, Whz1, bhz, Wxr0, Wxr1, bxr,
           Whr0, Whr1, bhr, Wxh0, Wxh1, bxh, Whh0, Whh1, bhh, Wl, bl,
           edge_index):
    e = edge_weight.shape[0]
    pad = _EPAD - e
    row = jnp.concatenate([edge_index[0], jnp.zeros((pad,), jnp.int32)])
    col = jnp.concatenate([edge_index[1], jnp.zeros((pad,), jnp.int32)])
    ew = jnp.concatenate([edge_weight, jnp.zeros((pad,), jnp.float32)])
    row3 = row.reshape(_NCHUNK, _NBLK, _BS)
    col3 = col.reshape(_NCHUNK, _NBLK, _BS)
    ew3 = ew.reshape(_NCHUNK, _NBLK, _BS)
    xpad = jnp.pad(x, ((0, _NPAD - x.shape[0]), (0, 0)))

    degp = _sc_degree(row3, ew3)
    degp_t = degp.T
    xs = _prescale(xpad, degp_t)
    parts = _sc_aggregate(xs, row.reshape(-1, _BS), col.reshape(-1, _BS),
                          ew.reshape(-1, _BS))

    bz = (bxz + bhz).reshape(1, -1)
    bh = (bxh + bhh).reshape(1, -1)
    out = _dense(xpad, parts[0], parts[1], degp_t, Wxz0, Wxz1, Wxh0, Wxh1,
                 bz, bh, Wl, bl.reshape(1, 1))
    return out[:_N]
